# baseline (speedup 1.0000x reference)
import functools

import jax
import jax.numpy as jnp
from jax import lax
from jax.experimental import pallas as pl
from jax.experimental.pallas import tpu as pltpu
from jax.experimental.pallas import tpu_sc as plsc

_N = 10000
_E = 320000
_D = 128
_H = 128

_NC = 2
_NS = 16
_NW = _NC * _NS
_EPW = _E // _NW
_CH = 128
_NCHUNK = 79
_PADW = 10240
_IB = 4
_NBATCH = _PADW // _CH // _IB
_DUMMY = 10200
_NP = 10240
_RPT = _NP // _NS


def _prop_body(h_hbm, srcr_hbm, dstr_hbm, out_hbm, sidx, didx, rows, acc,
               semg, sems, semi):
    c = lax.axis_index("c")
    s = lax.axis_index("s")
    wid = s * _NC + c

    zv = jnp.zeros((16,), jnp.float32)

    def _zrow(r, carry):
        for q in range(8):
            rows[0, r, pl.ds(q * 16, 16)] = zv
        return carry

    lax.fori_loop(0, _CH, _zrow, 0)
    for i in range(_RPT // _CH):
        base = s * _RPT + i * _CH
        pltpu.sync_copy(rows.at[0], acc.at[pl.ds(base, _CH)])

    pltpu.sync_copy(srcr_hbm.at[wid, 0], sidx.at[0])
    pltpu.sync_copy(dstr_hbm.at[wid, 0], didx.at[0])
    pltpu.async_copy(srcr_hbm.at[wid, 1], sidx.at[1], semi)
    pltpu.async_copy(dstr_hbm.at[wid, 1], didx.at[1], semi)
    plsc.subcore_barrier()

    pltpu.async_copy(h_hbm.at[sidx.at[0, 0]], rows.at[0], semg)
    pltpu.async_copy(h_hbm.at[sidx.at[0, 1]], rows.at[1], semg)

    def _pair(j2, carry):
        for b in range(2):
            j = 2 * j2 + b
            nb = j // _IB
            kk = lax.rem(j, _IB)
            p = lax.rem(nb, 2)
            pltpu.make_async_copy(h_hbm.at[sidx.at[p, kk]], rows.at[b],
                                  semg).wait()
            pltpu.async_copy(rows.at[b], acc.at[didx.at[p, kk]], sems,
                             add=True)
            pltpu.make_async_copy(rows.at[b], acc.at[didx.at[p, kk]],
                                  sems).wait()

            @pl.when(jnp.logical_and(kk == _IB - 2, nb <= _NBATCH - 2))
            def _():
                pltpu.make_async_copy(srcr_hbm.at[wid, nb + 1],
                                      sidx.at[1 - p], semi).wait()
                pltpu.make_async_copy(dstr_hbm.at[wid, nb + 1],
                                      didx.at[1 - p], semi).wait()

            @pl.when(jnp.logical_and(kk == _IB - 1, nb <= _NBATCH - 3))
            def _():
                pltpu.async_copy(srcr_hbm.at[wid, nb + 2], sidx.at[p],
                                 semi)
                pltpu.async_copy(dstr_hbm.at[wid, nb + 2], didx.at[p],
                                 semi)

            jn = j + 2
            nbn = jn // _IB
            pltpu.async_copy(
                h_hbm.at[sidx.at[lax.rem(nbn, 2), lax.rem(jn, _IB)]],
                rows.at[b], semg)
        return carry

    lax.fori_loop(0, (_NCHUNK - 1) // 2, _pair, 0)
    pltpu.make_async_copy(h_hbm.at[sidx.at[1, 2]], rows.at[0], semg).wait()
    pltpu.make_async_copy(h_hbm.at[sidx.at[1, 3]], rows.at[1], semg).wait()
    pltpu.sync_copy(rows.at[0], acc.at[didx.at[1, 2]], add=True)
    plsc.subcore_barrier()

    pltpu.sync_copy(acc.at[pl.ds(s * _RPT, _RPT)],
                    out_hbm.at[c, pl.ds(s * _RPT, _RPT)])


@functools.cache
def _get_prop():
    return pl.kernel(
        _prop_body,
        out_type=jax.ShapeDtypeStruct((_NC, _NP, _H), jnp.float32),
        mesh=plsc.VectorSubcoreMesh(core_axis_name="c", subcore_axis_name="s"),
        scratch_types=[
            pltpu.VMEM((2, _IB, _CH), jnp.int32),
            pltpu.VMEM((2, _IB, _CH), jnp.int32),
            pltpu.VMEM((2, _CH, _H), jnp.float32),
            pltpu.VMEM_SHARED((_NP, _H), jnp.float32),
            pltpu.SemaphoreType.DMA,
            pltpu.SemaphoreType.DMA,
            pltpu.SemaphoreType.DMA,
        ],
    )


def _tc1(h_ref, p_ref, w0_ref, w1_ref, h1_ref, acc_ref):
    h1 = p_ref[0] + p_ref[1]
    h1_ref[...] = h1
    acc_ref[...] = (
        jnp.dot(h_ref[...], w0_ref[...], preferred_element_type=jnp.float32)
        + jnp.dot(h1, w1_ref[...], preferred_element_type=jnp.float32))


_tc1_call = pl.pallas_call(
    _tc1,
    out_shape=(jax.ShapeDtypeStruct((_N, _H), jnp.float32),
               jax.ShapeDtypeStruct((_N, _H), jnp.float32)),
)


def _tc2(acc_ref, q_ref, w2_ref, b_ref, g_ref, be_ref, t_ref, h_ref):
    h2 = q_ref[0] + q_ref[1]
    out = (acc_ref[...]
           + jnp.dot(h2, w2_ref[...], preferred_element_type=jnp.float32)
           + b_ref[...])
    m = jnp.mean(out, axis=0, keepdims=True)
    d = out - m
    v = jnp.mean(d * d, axis=0, keepdims=True)
    xb = d * lax.rsqrt(v + 1e-5) * g_ref[...] + be_ref[...]
    half = _H // 2
    k = lax.broadcasted_iota(jnp.int32, (1, half), 1).astype(jnp.float32)
    freqs = jnp.exp(-jnp.log(10000.0) * k / (half - 1))
    args = t_ref[...] * freqs
    te = jnp.concatenate([jnp.sin(args), jnp.cos(args)], axis=-1)
    y = xb + te
    h_ref[...] = jnp.where(y > 0, y, 0.01 * y)


_tc2_call = pl.pallas_call(
    _tc2,
    out_shape=jax.ShapeDtypeStruct((_N, _H), jnp.float32),
)


def _tc_final(h_ref, w_ref, b_ref, y_ref):
    y_ref[...] = (
        jnp.dot(h_ref[...], w_ref[...], preferred_element_type=jnp.float32)
        + b_ref[...])


_tc_final_call = pl.pallas_call(
    _tc_final,
    out_shape=jax.ShapeDtypeStruct((_N, _D), jnp.float32),
)


def kernel(x, edge_index, t, conv0_W, conv0_b, conv1_W, conv1_b, conv2_W,
           conv2_b, bn_gamma, bn_beta, out_W, out_b):
    pad = _PADW - _EPW
    src_r = jnp.pad(edge_index[0].reshape(_NW, _EPW),
                    ((0, 0), (0, pad))).reshape(_NW, _NBATCH, _IB, _CH)
    dst_r = jnp.pad(edge_index[1].reshape(_NW, _EPW), ((0, 0), (0, pad)),
                    constant_values=_DUMMY).reshape(_NW, _NBATCH, _IB, _CH)
    t2 = t.reshape(_N, 1)
    convs = [(conv0_W, conv0_b), (conv1_W, conv1_b), (conv2_W, conv2_b)]
    prop = _get_prop()
    h = x
    for i, (W, b) in enumerate(convs):
        p = prop(h, src_r, dst_r)[:, :_N]
        h1, acc = _tc1_call(h, p, W[0], W[1])
        q = prop(h1, src_r, dst_r)[:, :_N]
        h = _tc2_call(acc, q, W[2], b.reshape(1, _H),
                      bn_gamma[i].reshape(1, _H), bn_beta[i].reshape(1, _H),
                      t2)
    return _tc_final_call(h, out_W, out_b.reshape(1, _D))


# speedup vs baseline: 2.1890x; 2.1890x over previous
import functools

import jax
import jax.numpy as jnp
from jax import lax
from jax.experimental import pallas as pl
from jax.experimental.pallas import tpu as pltpu
from jax.experimental.pallas import tpu_sc as plsc

_N = 10000
_E = 320000
_D = 128
_H = 128

_NC = 2
_NS = 16
_NW = _NC * _NS
_EPW = _E // _NW
_CH = 80
_IB = 25
_NB = 5
_NP = 10240
_RPT = _NP // _NS


def _prop_body(h_hbm, srcr_hbm, dstr_hbm, out_hbm, sidx, didx, rows, acc,
               semg):
    c = lax.axis_index("c")
    s = lax.axis_index("s")
    wid = s * _NC + c

    zv = jnp.zeros((16,), jnp.float32)

    def _zrow(r, carry):
        for q in range(8):
            rows[0, r, pl.ds(q * 16, 16)] = zv
        return carry

    lax.fori_loop(0, _CH, _zrow, 0)
    for i in range(_RPT // _CH):
        base = s * _RPT + i * _CH
        pltpu.sync_copy(rows.at[0], acc.at[pl.ds(base, _CH)])
    plsc.subcore_barrier()

    for nb in range(_NB):
        pltpu.sync_copy(srcr_hbm.at[wid, nb], sidx)
        pltpu.sync_copy(dstr_hbm.at[wid, nb], didx)
        pltpu.async_copy(h_hbm.at[sidx.at[0]], rows.at[0], semg)

        def _pair(k2, carry):
            for b in range(2):
                k = 2 * k2 + b
                pltpu.make_async_copy(h_hbm.at[sidx.at[k]], rows.at[b],
                                      semg).wait()
                pltpu.async_copy(h_hbm.at[sidx.at[k + 1]], rows.at[1 - b],
                                 semg)
                pltpu.sync_copy(rows.at[b], acc.at[didx.at[k]], add=True)
            return carry

        lax.fori_loop(0, _IB // 2, _pair, 0)
        pltpu.make_async_copy(h_hbm.at[sidx.at[_IB - 1]], rows.at[0],
                              semg).wait()
        pltpu.sync_copy(rows.at[0], acc.at[didx.at[_IB - 1]], add=True)
    plsc.subcore_barrier()

    pltpu.sync_copy(acc.at[pl.ds(s * _RPT, _RPT)],
                    out_hbm.at[c, pl.ds(s * _RPT, _RPT)])


@functools.cache
def _get_prop():
    return pl.kernel(
        _prop_body,
        out_type=jax.ShapeDtypeStruct((_NC, _NP, _H), jnp.float32),
        mesh=plsc.VectorSubcoreMesh(core_axis_name="c", subcore_axis_name="s"),
        scratch_types=[
            pltpu.VMEM((_IB, _CH), jnp.int32),
            pltpu.VMEM((_IB, _CH), jnp.int32),
            pltpu.VMEM((2, _CH, _H), jnp.float32),
            pltpu.VMEM_SHARED((_NP, _H), jnp.float32),
            pltpu.SemaphoreType.DMA,
        ],
    )


def _tc1(h_ref, p_ref, w0_ref, w1_ref, h1_ref, acc_ref):
    h1 = p_ref[0] + p_ref[1]
    h1_ref[...] = h1
    acc_ref[...] = (
        jnp.dot(h_ref[...], w0_ref[...], preferred_element_type=jnp.float32)
        + jnp.dot(h1, w1_ref[...], preferred_element_type=jnp.float32))


_tc1_call = pl.pallas_call(
    _tc1,
    out_shape=(jax.ShapeDtypeStruct((_N, _H), jnp.float32),
               jax.ShapeDtypeStruct((_N, _H), jnp.float32)),
)


def _tc2(acc_ref, q_ref, w2_ref, b_ref, g_ref, be_ref, t_ref, h_ref):
    h2 = q_ref[0] + q_ref[1]
    out = (acc_ref[...]
           + jnp.dot(h2, w2_ref[...], preferred_element_type=jnp.float32)
           + b_ref[...])
    m = jnp.mean(out, axis=0, keepdims=True)
    d = out - m
    v = jnp.mean(d * d, axis=0, keepdims=True)
    xb = d * lax.rsqrt(v + 1e-5) * g_ref[...] + be_ref[...]
    half = _H // 2
    k = lax.broadcasted_iota(jnp.int32, (1, half), 1).astype(jnp.float32)
    freqs = jnp.exp(-jnp.log(10000.0) * k / (half - 1))
    args = t_ref[...] * freqs
    te = jnp.concatenate([jnp.sin(args), jnp.cos(args)], axis=-1)
    y = xb + te
    h_ref[...] = jnp.where(y > 0, y, 0.01 * y)


_tc2_call = pl.pallas_call(
    _tc2,
    out_shape=jax.ShapeDtypeStruct((_N, _H), jnp.float32),
)


def _tc_final(h_ref, w_ref, b_ref, y_ref):
    y_ref[...] = (
        jnp.dot(h_ref[...], w_ref[...], preferred_element_type=jnp.float32)
        + b_ref[...])


_tc_final_call = pl.pallas_call(
    _tc_final,
    out_shape=jax.ShapeDtypeStruct((_N, _D), jnp.float32),
)


def kernel(x, edge_index, t, conv0_W, conv0_b, conv1_W, conv1_b, conv2_W,
           conv2_b, bn_gamma, bn_beta, out_W, out_b):
    src_r = edge_index[0].reshape(_NW, _NB, _IB, _CH)
    dst_r = edge_index[1].reshape(_NW, _NB, _IB, _CH)
    t2 = t.reshape(_N, 1)
    convs = [(conv0_W, conv0_b), (conv1_W, conv1_b), (conv2_W, conv2_b)]
    prop = _get_prop()
    h = x
    for i, (W, b) in enumerate(convs):
        p = prop(h, src_r, dst_r)[:, :_N]
        h1, acc = _tc1_call(h, p, W[0], W[1])
        q = prop(h1, src_r, dst_r)[:, :_N]
        h = _tc2_call(acc, q, W[2], b.reshape(1, _H),
                      bn_gamma[i].reshape(1, _H), bn_beta[i].reshape(1, _H),
                      t2)
    return _tc_final_call(h, out_W, out_b.reshape(1, _D))
